# baseline (speedup 1.0000x reference)
import functools

import jax
import jax.numpy as jnp
from jax import lax
from jax.experimental import pallas as pl
from jax.experimental.pallas import tpu as pltpu
from jax.experimental.pallas import tpu_sc as plsc

_EPS = 1e-5
_B = 16384
_D = 64
_NC = 2
_NS = 16
_NW = _NC * _NS
_BPW = _B // _NW


def _make_sc_gather():
    mesh = plsc.VectorSubcoreMesh(core_axis_name="c", subcore_axis_name="s")
    out_t = tuple(
        jax.ShapeDtypeStruct((_B, _D), jnp.float32) for _ in range(4)
    )

    @functools.partial(
        pl.kernel,
        mesh=mesh,
        out_type=out_t,
        compiler_params=pltpu.CompilerParams(use_tc_tiling_on_sc=False),
        scratch_types=[
            pltpu.VMEM((_BPW,), jnp.int32),
            pltpu.VMEM((_BPW,), jnp.int32),
            pltpu.VMEM((_BPW, _D), jnp.float32),
            pltpu.VMEM((_BPW, _D), jnp.float32),
            pltpu.SemaphoreType.DMA,
            pltpu.SemaphoreType.DMA,
        ],
    )
    def sc_gather(uid_hbm, iid_hbm, ug_hbm, ig_hbm, um_hbm, im_hbm,
                  oug, oig, oum, oim,
                  uidx, iidx, bufa, bufb, sema, semb):
        wid = lax.axis_index("s") * _NC + lax.axis_index("c")
        base = wid * _BPW
        pltpu.sync_copy(uid_hbm.at[pl.ds(base, _BPW)], uidx)
        pltpu.sync_copy(iid_hbm.at[pl.ds(base, _BPW)], iidx)
        ca = pltpu.async_copy(ug_hbm.at[uidx], bufa, sema)
        cb = pltpu.async_copy(ig_hbm.at[iidx], bufb, semb)
        ca.wait()
        pltpu.sync_copy(bufa, oug.at[pl.ds(base, _BPW)])
        ca = pltpu.async_copy(um_hbm.at[uidx], bufa, sema)
        cb.wait()
        pltpu.sync_copy(bufb, oig.at[pl.ds(base, _BPW)])
        cb = pltpu.async_copy(im_hbm.at[iidx], bufb, semb)
        ca.wait()
        pltpu.sync_copy(bufa, oum.at[pl.ds(base, _BPW)])
        cb.wait()
        pltpu.sync_copy(bufb, oim.at[pl.ds(base, _BPW)])

    return sc_gather


_SC_GATHER_CACHE = []


def _sc_gather(*args):
    if not _SC_GATHER_CACHE:
        _SC_GATHER_CACHE.append(_make_sc_gather())
    return _SC_GATHER_CACHE[0](*args)

_BLK = 2048


def _tc_body(ug_r, ig_r, um_r, im_r,
             w1u_r, w1i_r, b1_r, s1_r, e1_r,
             w2_r, b2_r, s2_r, e2_r,
             w3_r, b3_r, s3_r, e3_r,
             wfg_r, wfh_r, bf_r, out_r):
    h = jnp.dot(um_r[...], w1u_r[...], preferred_element_type=jnp.float32)
    h += jnp.dot(im_r[...], w1i_r[...], preferred_element_type=jnp.float32)
    h = jnp.maximum(h + b1_r[...], 0.0) * s1_r[...] + e1_r[...]
    h = jnp.dot(h, w2_r[...], preferred_element_type=jnp.float32)
    h = jnp.maximum(h + b2_r[...], 0.0) * s2_r[...] + e2_r[...]
    h = jnp.dot(h, w3_r[...], preferred_element_type=jnp.float32)
    h = jnp.maximum(h + b3_r[...], 0.0) * s3_r[...] + e3_r[...]
    g = ug_r[...] * ig_r[...]
    logit = jnp.sum(g * wfg_r[...] + h * wfh_r[...], axis=1) + bf_r[0, 0]
    out_r[...] = jax.nn.sigmoid(logit)


def _tc_forward(ug, ig, um, im, w1u, w1i, b1, s1, e1,
                w2, b2, s2, e2, w3, b3, s3, e3, wfg, wfh, bf):
    n_blk = _B // _BLK
    row_spec = pl.BlockSpec((_BLK, _D), lambda i: (i, 0))

    def full(shape):
        return pl.BlockSpec(shape, lambda i: tuple(0 for _ in shape))

    return pl.pallas_call(
        _tc_body,
        grid=(n_blk,),
        in_specs=[
            row_spec, row_spec, row_spec, row_spec,
            full((_D, 256)), full((_D, 256)), full((1, 256)), full((1, 256)),
            full((1, 256)),
            full((256, 128)), full((1, 128)), full((1, 128)), full((1, 128)),
            full((128, 64)), full((1, 64)), full((1, 64)), full((1, 64)),
            full((1, _D)), full((1, _D)), full((1, 1)),
        ],
        out_specs=pl.BlockSpec((_BLK,), lambda i: (i,)),
        out_shape=jax.ShapeDtypeStruct((_B,), jnp.float32),
    )(ug, ig, um, im, w1u, w1i, b1, s1, e1,
      w2, b2, s2, e2, w3, b3, s3, e3, wfg, wfh, bf)


def kernel(user_ids, item_ids, user_gmf, item_gmf, user_mlp, item_mlp,
           W1, b1, g1, be1, W2, b2, g2, be2, W3, b3, g3, be3, Wf, bf):
    ug, ig, um, im = _sc_gather(user_ids, item_ids, user_gmf, item_gmf,
                                user_mlp, item_mlp)

    inv = 1.0 / jnp.sqrt(jnp.float32(1.0) + jnp.float32(_EPS))
    s1 = (g1 * inv).reshape(1, -1)
    s2 = (g2 * inv).reshape(1, -1)
    s3 = (g3 * inv).reshape(1, -1)
    return _tc_forward(
        ug, ig, um, im,
        W1[:_D], W1[_D:], b1.reshape(1, -1), s1, be1.reshape(1, -1),
        W2, b2.reshape(1, -1), s2, be2.reshape(1, -1),
        W3, b3.reshape(1, -1), s3, be3.reshape(1, -1),
        Wf[:_D, 0].reshape(1, -1), Wf[_D:, 0].reshape(1, -1),
        bf.reshape(1, 1),
    )


# speedup vs baseline: 1.2622x; 1.2622x over previous
import functools

import jax
import jax.numpy as jnp
from jax import lax
from jax.experimental import pallas as pl
from jax.experimental.pallas import tpu as pltpu
from jax.experimental.pallas import tpu_sc as plsc

_EPS = 1e-5
_B = 16384
_D = 64
_F = 2 * _D
_NC = 2
_NS = 16
_NW = _NC * _NS
_BPW = _B // _NW
_CH = _BPW // 2


def _make_sc_gather():
    mesh = plsc.VectorSubcoreMesh(core_axis_name="c", subcore_axis_name="s")

    @functools.partial(
        pl.kernel,
        mesh=mesh,
        out_type=jax.ShapeDtypeStruct((_B, _F), jnp.float32),
        scratch_types=[
            pltpu.VMEM((_CH,), jnp.int32),
            pltpu.VMEM((_CH,), jnp.int32),
            pltpu.VMEM((_CH, _F), jnp.float32),
            pltpu.VMEM((_CH, _F), jnp.float32),
            pltpu.SemaphoreType.DMA,
            pltpu.SemaphoreType.DMA,
        ],
    )
    def sc_gather(tbl_hbm, ids_hbm, out_hbm, idxa, idxb, bufa, bufb,
                  sema, semb):
        wid = lax.axis_index("s") * _NC + lax.axis_index("c")
        base = wid * _BPW
        pltpu.sync_copy(ids_hbm.at[pl.ds(base, _CH)], idxa)
        pltpu.sync_copy(ids_hbm.at[pl.ds(base + _CH, _CH)], idxb)
        ca = pltpu.async_copy(tbl_hbm.at[idxa], bufa, sema)
        cb = pltpu.async_copy(tbl_hbm.at[idxb], bufb, semb)
        ca.wait()
        pltpu.sync_copy(bufa, out_hbm.at[pl.ds(base, _CH)])
        cb.wait()
        pltpu.sync_copy(bufb, out_hbm.at[pl.ds(base + _CH, _CH)])

    return sc_gather


_SC_GATHER_CACHE = []


def _sc_gather(tbl, ids):
    if not _SC_GATHER_CACHE:
        _SC_GATHER_CACHE.append(_make_sc_gather())
    return _SC_GATHER_CACHE[0](tbl, ids)


_BLK = 2048


def _tc_body(u_r, i_r,
             w1u_r, w1i_r, b1_r, s1_r, e1_r,
             w2_r, b2_r, s2_r, e2_r,
             w3_r, b3_r, s3_r, e3_r,
             wfg_r, wfh_r, bf_r, out_r):
    um = u_r[:, _D:]
    im = i_r[:, _D:]
    h = jnp.dot(um, w1u_r[...], preferred_element_type=jnp.float32)
    h += jnp.dot(im, w1i_r[...], preferred_element_type=jnp.float32)
    h = jnp.maximum(h + b1_r[...], 0.0) * s1_r[...] + e1_r[...]
    h = jnp.dot(h, w2_r[...], preferred_element_type=jnp.float32)
    h = jnp.maximum(h + b2_r[...], 0.0) * s2_r[...] + e2_r[...]
    h = jnp.dot(h, w3_r[...], preferred_element_type=jnp.float32)
    h = jnp.maximum(h + b3_r[...], 0.0) * s3_r[...] + e3_r[...]
    g = u_r[:, :_D] * i_r[:, :_D]
    logit = jnp.sum(g * wfg_r[...] + h * wfh_r[...], axis=1) + bf_r[0, 0]
    out_r[...] = jax.nn.sigmoid(logit)


def _tc_forward(rows_u, rows_i, w1u, w1i, b1, s1, e1,
                w2, b2, s2, e2, w3, b3, s3, e3, wfg, wfh, bf):
    n_blk = _B // _BLK
    row_spec = pl.BlockSpec((_BLK, _F), lambda i: (i, 0))

    def full(shape):
        return pl.BlockSpec(shape, lambda i: tuple(0 for _ in shape))

    return pl.pallas_call(
        _tc_body,
        grid=(n_blk,),
        in_specs=[
            row_spec, row_spec,
            full((_D, 256)), full((_D, 256)), full((1, 256)), full((1, 256)),
            full((1, 256)),
            full((256, 128)), full((1, 128)), full((1, 128)), full((1, 128)),
            full((128, 64)), full((1, 64)), full((1, 64)), full((1, 64)),
            full((1, _D)), full((1, _D)), full((1, 1)),
        ],
        out_specs=pl.BlockSpec((_BLK,), lambda i: (i,)),
        out_shape=jax.ShapeDtypeStruct((_B,), jnp.float32),
    )(rows_u, rows_i, w1u, w1i, b1, s1, e1,
      w2, b2, s2, e2, w3, b3, s3, e3, wfg, wfh, bf)


def kernel(user_ids, item_ids, user_gmf, item_gmf, user_mlp, item_mlp,
           W1, b1, g1, be1, W2, b2, g2, be2, W3, b3, g3, be3, Wf, bf):
    uf = jnp.concatenate([user_gmf, user_mlp], axis=1)
    itf = jnp.concatenate([item_gmf, item_mlp], axis=1)
    rows_u = _sc_gather(uf, user_ids)
    rows_i = _sc_gather(itf, item_ids)

    inv = 1.0 / jnp.sqrt(jnp.float32(1.0) + jnp.float32(_EPS))
    s1 = (g1 * inv).reshape(1, -1)
    s2 = (g2 * inv).reshape(1, -1)
    s3 = (g3 * inv).reshape(1, -1)
    return _tc_forward(
        rows_u, rows_i,
        W1[:_D], W1[_D:], b1.reshape(1, -1), s1, be1.reshape(1, -1),
        W2, b2.reshape(1, -1), s2, be2.reshape(1, -1),
        W3, b3.reshape(1, -1), s3, be3.reshape(1, -1),
        Wf[:_D, 0].reshape(1, -1), Wf[_D:, 0].reshape(1, -1),
        bf.reshape(1, 1),
    )
